# baseline (speedup 1.0000x reference)
import functools

import jax
import jax.numpy as jnp
from jax import lax
from jax.experimental import pallas as pl
from jax.experimental.pallas import tpu as pltpu
from jax.experimental.pallas import tpu_sc as plsc

_KEEP_RATE = 0.8
_EPS = 1e-5
_BS = 512
_L = 16
_NC = 2


def _scores_body(x_ref, s_ref):
    x = x_ref[...]
    s = jnp.sqrt(jnp.sum(x * x, axis=-1))[:, None, :]
    s_ref[...] = lax.bitcast_convert_type(s, jnp.int32)


def _ln_body(x_ref, m_ref, g_ref, b_ref, o_ref):
    x = x_ref[0]
    m = m_ref[...]
    masked = x * m
    mu = jnp.mean(masked, axis=-1, keepdims=True)
    var = jnp.mean((masked - mu) ** 2, axis=-1, keepdims=True)
    xhat = (masked - mu) / jnp.sqrt(var + _EPS)
    o_ref[0] = xhat * g_ref[...] + b_ref[...]


def _scan_vreg(h, cum_above, need, iota):
    rev = lax.rev(h, (0,))
    cs = plsc.cumsum(rev)
    cse = cs - rev
    above = cum_above + cse
    cross = ((cum_above + cs) >= need) & (above < need)
    crossi = cross.astype(jnp.int32)
    anyv = jnp.sum(crossi)
    lane = jnp.sum(jnp.where(cross, iota, 0))
    boff = 15 - lane
    n_above = jnp.sum(jnp.where(cross, above, 0))
    return anyv, boff, n_above


def _sc_level(scores_v, hist_v, coarse_v, nv, shift, nbits, pmask, prefix,
              n_gt, keep_k):
    nbuck = 1 << nbits
    ncoarse = nbuck // _L
    zeros = jnp.zeros((_L,), jnp.int32)
    ones = jnp.ones((_L,), jnp.int32)
    iota = lax.iota(jnp.int32, _L)

    def zf(j, c):
        hist_v[pl.ds(j * _L, _L)] = zeros
        return c
    lax.fori_loop(0, nbuck // _L, zf, 0)

    def zc(j, c):
        coarse_v[pl.ds(j * _L, _L)] = zeros
        return c
    lax.fori_loop(0, ncoarse // _L, zc, 0)

    def acc(j, c):
        b = scores_v[pl.ds(j * _L, _L)]
        inr = (b & pmask) == prefix
        buck = (b >> shift) & (nbuck - 1)
        plsc.addupdate_scatter(hist_v, [buck], ones, mask=inr)
        plsc.addupdate_scatter(coarse_v, [buck >> 4], ones, mask=inr)
        return c
    lax.fori_loop(0, nv, acc, 0)

    need = keep_k - n_gt

    def cscan(jj, carry):
        found, g_star, n_above, cum = carry
        g = ncoarse // _L - 1 - jj
        h = coarse_v[pl.ds(g * _L, _L)]
        anyv, boff, na = _scan_vreg(h, cum, need, iota)
        hit = (anyv > 0) & (found == 0)
        g_star = jnp.where(hit, g * _L + boff, g_star)
        n_above = jnp.where(hit, na, n_above)
        found = found | anyv
        cum = cum + jnp.sum(h)
        return found, g_star, n_above, cum

    init = (jnp.int32(0), jnp.int32(0), jnp.int32(0), jnp.int32(0))
    _, g_star, n_above_c, _ = lax.fori_loop(0, ncoarse // _L, cscan, init)

    hf = plsc.load_gather(hist_v, [g_star * _L + iota])
    _, boff, n_above_f = _scan_vreg(hf, n_above_c, need, iota)
    bucket = g_star * _L + boff
    n_gt_new = n_gt + n_above_f
    prefix_new = prefix | (bucket << shift)
    return prefix_new, n_gt_new


def _sc_mask_body(s_hbm, m_hbm, scores_v, mask_v, hist_v, coarse_v, *,
                  keep_k, seq, batch):
    wid = lax.axis_index("s") * _NC + lax.axis_index("c")

    @pl.when(wid < batch)
    def _():
        pltpu.sync_copy(s_hbm.at[wid], scores_v)
        nv = seq // _L
        prefix, n_gt = jnp.int32(0), jnp.int32(0)
        prefix, n_gt = _sc_level(scores_v, hist_v, coarse_v, nv, 20, 11,
                                 jnp.int32(0), prefix, n_gt, keep_k)
        prefix, n_gt = _sc_level(scores_v, hist_v, coarse_v, nv, 10, 10,
                                 jnp.int32(0x7FF00000), prefix, n_gt, keep_k)
        prefix, n_gt = _sc_level(scores_v, hist_v, coarse_v, nv, 0, 10,
                                 jnp.int32(0x7FFFFC00), prefix, n_gt, keep_k)
        thresh = prefix
        need_eq = keep_k - n_gt

        def mk(j, run):
            b = scores_v[pl.ds(j * _L, _L)]
            gt = b > thresh
            eq = b == thresh
            eqi = eq.astype(jnp.int32)
            cs = plsc.cumsum(eqi)
            keep_eq = eq & ((run + cs) <= need_eq)
            mask_v[pl.ds(j * _L, _L)] = jnp.where(gt | keep_eq, 1.0, 0.0)
            return run + jnp.sum(eqi)
        lax.fori_loop(0, nv, mk, jnp.int32(0))
        pltpu.sync_copy(mask_v, m_hbm.at[pl.ds(wid * seq, seq)])


def kernel(hidden_states, gamma, beta):
    batch, seq, dim = hidden_states.shape
    keep_k = max(1, int(seq * _KEEP_RATE))
    bs = min(_BS, seq)
    nblk = (batch * seq) // bs
    x3 = hidden_states.reshape(nblk, bs, dim)

    scores = pl.pallas_call(
        _scores_body,
        grid=(nblk,),
        in_specs=[pl.BlockSpec((1, bs, dim), lambda i: (i, 0, 0))],
        out_specs=pl.BlockSpec((1, 1, bs), lambda i: (i, 0, 0)),
        out_shape=jax.ShapeDtypeStruct((nblk, 1, bs), jnp.int32),
    )(x3)
    scores2 = scores.reshape(batch, seq)

    mesh = plsc.VectorSubcoreMesh(core_axis_name="c", subcore_axis_name="s")
    mask_flat = pl.kernel(
        functools.partial(_sc_mask_body, keep_k=keep_k, seq=seq, batch=batch),
        out_type=jax.ShapeDtypeStruct((batch * seq,), jnp.float32),
        mesh=mesh,
        compiler_params=pltpu.CompilerParams(needs_layout_passes=False),
        scratch_types=[
            pltpu.VMEM((seq,), jnp.int32),
            pltpu.VMEM((seq,), jnp.float32),
            pltpu.VMEM((2048,), jnp.int32),
            pltpu.VMEM((128,), jnp.int32),
        ],
    )(scores2)

    mask_col = mask_flat.reshape(batch * seq, 1)
    out = pl.pallas_call(
        _ln_body,
        grid=(nblk,),
        in_specs=[
            pl.BlockSpec((1, bs, dim), lambda i: (i, 0, 0)),
            pl.BlockSpec((bs, 1), lambda i: (i, 0)),
            pl.BlockSpec((dim,), lambda i: (0,)),
            pl.BlockSpec((dim,), lambda i: (0,)),
        ],
        out_specs=pl.BlockSpec((1, bs, dim), lambda i: (i, 0, 0)),
        out_shape=jax.ShapeDtypeStruct((nblk, bs, dim), jnp.float32),
    )(x3, mask_col, gamma, beta)
    return out.reshape(batch, seq, dim)
